# Initial kernel scaffold; baseline (speedup 1.0000x reference)
import functools

import jax
import jax.numpy as jnp
from jax import lax
from jax.experimental import pallas as pl
from jax.experimental.pallas import tpu as pltpu
from jax.experimental.pallas import tpu_sc as plsc

N = 10000
D = 128
E = 320000

NC = 2
NS = 16
NACC = 10240
RT = NACC // NS
CT = 80
EPAD = NC * NS * CT * 128
ECHUNKS = EPAD // 128

_MESH = plsc.VectorSubcoreMesh(core_axis_name="c", subcore_axis_name="s")


def _zero_fill(ref, nrows, ncols):
  zv = jnp.zeros((16,), jnp.float32)

  def row(i, _):
    def col(j, _):
      ref[i, pl.ds(j * 16, 16)] = zv
      return 0
    return lax.fori_loop(0, ncols // 16, col, 0)

  lax.fori_loop(0, nrows, row, 0)


@functools.partial(
    pl.kernel,
    out_type=jax.ShapeDtypeStruct((NC, NACC, 16), jnp.float32),
    mesh=_MESH,
    scratch_types=[
        pltpu.VMEM_SHARED((NACC, 16), jnp.float32),
        pltpu.VMEM((CT, 128), jnp.int32),
        pltpu.VMEM((128, 16), jnp.float32),
        pltpu.VMEM((RT, 16), jnp.float32),
    ],
)
def _sc_degree(dst2_hbm, out_hbm, htab, idx_d, ones, zbuf):
  c = lax.axis_index("c")
  s = lax.axis_index("s")
  wid = s * NC + c

  ov = jnp.ones((16,), jnp.float32)

  def orow(i, _):
    ones[i, pl.ds(0, 16)] = ov
    return 0
  lax.fori_loop(0, 128, orow, 0)
  _zero_fill(zbuf, RT, 16)

  pltpu.sync_copy(zbuf, htab.at[pl.ds(s * RT, RT)])
  plsc.subcore_barrier()

  pltpu.sync_copy(dst2_hbm.at[pl.ds(wid * CT, CT)], idx_d)

  def chunk(k, _):
    pltpu.sync_copy(ones, htab.at[idx_d.at[k]], add=True)
    return 0
  lax.fori_loop(0, CT, chunk, 0)

  plsc.subcore_barrier()
  pltpu.sync_copy(htab.at[pl.ds(s * RT, RT)], zbuf)
  pltpu.sync_copy(zbuf, out_hbm.at[c, pl.ds(s * RT, RT)])


@functools.partial(
    pl.kernel,
    out_type=jax.ShapeDtypeStruct((NC, NACC, D), jnp.float32),
    mesh=_MESH,
    scratch_types=[
        pltpu.VMEM_SHARED((NACC, D), jnp.float32),
        pltpu.VMEM((CT, 128), jnp.int32),
        pltpu.VMEM((CT, 128), jnp.int32),
        pltpu.VMEM((128, D), jnp.float32),
        pltpu.VMEM((128, D), jnp.float32),
        pltpu.SemaphoreType.DMA,
    ],
)
def _sc_aggregate(z_hbm, src2_hbm, dst2_hbm, out_hbm,
                  acc, idx_s, idx_d, rows, zbuf, sem):
  c = lax.axis_index("c")
  s = lax.axis_index("s")
  wid = s * NC + c

  _zero_fill(zbuf, 128, D)
  for t in range(RT // 128):
    pltpu.sync_copy(zbuf, acc.at[pl.ds(s * RT + t * 128, 128)])
  plsc.subcore_barrier()

  pltpu.sync_copy(src2_hbm.at[pl.ds(wid * CT, CT)], idx_s)
  pltpu.sync_copy(dst2_hbm.at[pl.ds(wid * CT, CT)], idx_d)

  def chunk(k, _):
    pltpu.async_copy(z_hbm.at[idx_s.at[k]], rows, sem).wait()
    pltpu.sync_copy(rows, acc.at[idx_d.at[k]], add=True)
    return 0
  lax.fori_loop(0, CT, chunk, 0)

  plsc.subcore_barrier()
  for t in range(RT // 128):
    pltpu.sync_copy(acc.at[pl.ds(s * RT + t * 128, 128)], zbuf)
    pltpu.sync_copy(zbuf, out_hbm.at[c, pl.ds(s * RT + t * 128, 128)])


_RB = 2000


def _tc1_body(x_ref, d0_ref, d1_ref, w_ref, z_ref, dinv_ref):
  deg = d0_ref[:, 0:1] + d1_ref[:, 0:1] + 1.0
  dinv = lax.rsqrt(deg)
  xw = jnp.dot(x_ref[:, :], w_ref[:, :], preferred_element_type=jnp.float32)
  z_ref[:, :] = xw * dinv
  dinv_ref[:, :] = jnp.broadcast_to(dinv, (_RB, 16))


def _tc_z1(x, d0, d1, W1):
  return pl.pallas_call(
      _tc1_body,
      grid=(N // _RB,),
      in_specs=[
          pl.BlockSpec((_RB, D), lambda i: (i, 0)),
          pl.BlockSpec((_RB, 16), lambda i: (i, 0)),
          pl.BlockSpec((_RB, 16), lambda i: (i, 0)),
          pl.BlockSpec((D, D), lambda i: (0, 0)),
      ],
      out_specs=[
          pl.BlockSpec((_RB, D), lambda i: (i, 0)),
          pl.BlockSpec((_RB, 16), lambda i: (i, 0)),
      ],
      out_shape=[
          jax.ShapeDtypeStruct((N, D), jnp.float32),
          jax.ShapeDtypeStruct((N, 16), jnp.float32),
      ],
  )(x, d0, d1, W1)


def _tc2_body(a0_ref, a1_ref, z1_ref, dinv_ref, b_ref, w_ref, z2_ref):
  dinv = dinv_ref[:, 0:1]
  h = (a0_ref[:, :] + a1_ref[:, :] + z1_ref[:, :]) * dinv + b_ref[:, :]
  h = jnp.maximum(h, 0.0)
  z2_ref[:, :] = jnp.dot(h, w_ref[:, :],
                         preferred_element_type=jnp.float32) * dinv


def _tc_mid(a0, a1, z1, dinv16, b1, W2):
  return pl.pallas_call(
      _tc2_body,
      grid=(N // _RB,),
      in_specs=[
          pl.BlockSpec((_RB, D), lambda i: (i, 0)),
          pl.BlockSpec((_RB, D), lambda i: (i, 0)),
          pl.BlockSpec((_RB, D), lambda i: (i, 0)),
          pl.BlockSpec((_RB, 16), lambda i: (i, 0)),
          pl.BlockSpec((1, D), lambda i: (0, 0)),
          pl.BlockSpec((D, D), lambda i: (0, 0)),
      ],
      out_specs=pl.BlockSpec((_RB, D), lambda i: (i, 0)),
      out_shape=jax.ShapeDtypeStruct((N, D), jnp.float32),
  )(a0, a1, z1, dinv16, b1, W2)


def _tc3_body(a0_ref, a1_ref, z2_ref, dinv_ref, b_ref, g_ref, beta_ref,
              out_ref):
  dinv = dinv_ref[:, 0:1]
  h = (a0_ref[:, :] + a1_ref[:, :] + z2_ref[:, :]) * dinv + b_ref[:, :]
  mu = jnp.mean(h, axis=-1, keepdims=True)
  hc = h - mu
  var = jnp.mean(hc * hc, axis=-1, keepdims=True)
  out_ref[:, :] = hc * lax.rsqrt(var + 1e-5) * g_ref[:, :] + beta_ref[:, :]


def _tc_final(a0, a1, z2, dinv16, b2, gamma, beta):
  return pl.pallas_call(
      _tc3_body,
      grid=(N // _RB,),
      in_specs=[
          pl.BlockSpec((_RB, D), lambda i: (i, 0)),
          pl.BlockSpec((_RB, D), lambda i: (i, 0)),
          pl.BlockSpec((_RB, D), lambda i: (i, 0)),
          pl.BlockSpec((_RB, 16), lambda i: (i, 0)),
          pl.BlockSpec((1, D), lambda i: (0, 0)),
          pl.BlockSpec((1, D), lambda i: (0, 0)),
          pl.BlockSpec((1, D), lambda i: (0, 0)),
      ],
      out_specs=pl.BlockSpec((_RB, D), lambda i: (i, 0)),
      out_shape=jax.ShapeDtypeStruct((N, D), jnp.float32),
  )(a0, a1, z2, dinv16, b2, gamma, beta)


def kernel(x, edge_index, W1, b1, W2, b2, gamma, beta):
  src = edge_index[0]
  dst = edge_index[1]
  padi = jnp.full((EPAD - E,), N, jnp.int32)
  src2 = jnp.concatenate([src, padi]).reshape(ECHUNKS, 128)
  dst2 = jnp.concatenate([dst, padi]).reshape(ECHUNKS, 128)

  degp = _sc_degree(dst2)
  d0 = degp[0, :N, :]
  d1 = degp[1, :N, :]

  z1, dinv16 = _tc_z1(x, d0, d1, W1)

  zpad = jnp.zeros((NACC - N, D), jnp.float32)
  ag1 = _sc_aggregate(jnp.concatenate([z1, zpad]), src2, dst2)
  z2 = _tc_mid(ag1[0, :N], ag1[1, :N], z1, dinv16,
               b1.reshape(1, D), W2)

  ag2 = _sc_aggregate(jnp.concatenate([z2, zpad]), src2, dst2)
  return _tc_final(ag2[0, :N], ag2[1, :N], z2, dinv16,
                   b2.reshape(1, D), gamma.reshape(1, D),
                   beta.reshape(1, D))


# speedup vs baseline: 8.8248x; 8.8248x over previous
import functools

import jax
import jax.numpy as jnp
from jax import lax
from jax.experimental import pallas as pl
from jax.experimental.pallas import tpu as pltpu
from jax.experimental.pallas import tpu_sc as plsc

N = 10000
D = 128
E = 320000

NC = 2
NS = 16
NACC = 10240
RT = NACC // NS
CT = 80
EPAD = NC * NS * CT * 128
ECHUNKS = EPAD // 128

_MESH = plsc.VectorSubcoreMesh(core_axis_name="c", subcore_axis_name="s")


def _zero_fill(ref, nrows, ncols):
  zv = jnp.zeros((16,), jnp.float32)

  def row(i, _):
    def col(j, _):
      ref[i, pl.ds(j * 16, 16)] = zv
      return 0
    return lax.fori_loop(0, ncols // 16, col, 0)

  lax.fori_loop(0, nrows, row, 0)


@functools.partial(
    pl.kernel,
    out_type=jax.ShapeDtypeStruct((NC, NS, 8, 128), jnp.float32),
    mesh=_MESH,
    scratch_types=[
        pltpu.VMEM_SHARED((NACC, 128), jnp.float32),
        pltpu.VMEM((CT, 128), jnp.int32),
        pltpu.VMEM((128, 128), jnp.float32),
        pltpu.VMEM((128, 128), jnp.float32),
        pltpu.VMEM((8, 128), jnp.float32),
    ],
)
def _sc_degree(dst2_hbm, out_hbm, htab, idx_d, ones, zbuf, compact):
  c = lax.axis_index("c")
  s = lax.axis_index("s")
  wid = s * NC + c

  _zero_fill(ones, 128, 128)
  for t in range(RT // 128):
    pltpu.sync_copy(ones, htab.at[pl.ds(s * RT + t * 128, 128)])

  ov = jnp.ones((16,), jnp.float32)

  def orow(i, _):
    def ocol(j, _):
      ones[i, pl.ds(j * 16, 16)] = ov
      return 0
    return lax.fori_loop(0, 128 // 16, ocol, 0)
  lax.fori_loop(0, 128, orow, 0)
  plsc.subcore_barrier()

  pltpu.sync_copy(dst2_hbm.at[pl.ds(wid * CT, CT)], idx_d)

  def chunk(k, _):
    pltpu.sync_copy(ones, htab.at[idx_d.at[k]], add=True)
    return 0
  lax.fori_loop(0, CT, chunk, 0)

  plsc.subcore_barrier()
  iota16 = lax.iota(jnp.int32, 16)
  zf = jnp.zeros((16,), jnp.float32)
  for r in range(5, 8):
    for cc in range(8):
      compact[r, pl.ds(cc * 16, 16)] = zf
  for t in range(RT // 128):
    pltpu.sync_copy(htab.at[pl.ds(s * RT + t * 128, 128)], zbuf)
    for j8 in range(8):
      v = zf
      for i in range(16):
        rv = zbuf[j8 * 16 + i, pl.ds(0, 16)]
        v = v + jnp.where(iota16 == i, rv, 0.0)
      compact[t, pl.ds(j8 * 16, 16)] = v
  pltpu.sync_copy(compact, out_hbm.at[c, s])


@functools.partial(
    pl.kernel,
    out_type=jax.ShapeDtypeStruct((NC, NACC, D), jnp.float32),
    mesh=_MESH,
    scratch_types=[
        pltpu.VMEM_SHARED((NACC, D), jnp.float32),
        pltpu.VMEM((CT, 128), jnp.int32),
        pltpu.VMEM((CT, 128), jnp.int32),
        pltpu.VMEM((128, D), jnp.float32),
        pltpu.SemaphoreType.DMA,
    ],
)
def _sc_aggregate(z_hbm, src2_hbm, dst2_hbm, out_hbm,
                  acc, idx_s, idx_d, rows, sem):
  c = lax.axis_index("c")
  s = lax.axis_index("s")
  wid = s * NC + c

  _zero_fill(rows, 128, D)
  for t in range(RT // 128):
    pltpu.sync_copy(rows, acc.at[pl.ds(s * RT + t * 128, 128)])
  plsc.subcore_barrier()

  pltpu.sync_copy(src2_hbm.at[pl.ds(wid * CT, CT)], idx_s)
  pltpu.sync_copy(dst2_hbm.at[pl.ds(wid * CT, CT)], idx_d)

  def chunk(k, _):
    pltpu.async_copy(z_hbm.at[idx_s.at[k]], rows, sem).wait()
    pltpu.sync_copy(rows, acc.at[idx_d.at[k]], add=True)
    return 0
  lax.fori_loop(0, CT, chunk, 0)

  plsc.subcore_barrier()
  for t in range(RT // 128):
    pltpu.sync_copy(acc.at[pl.ds(s * RT + t * 128, 128)], rows)
    pltpu.sync_copy(rows, out_hbm.at[c, pl.ds(s * RT + t * 128, 128)])


_RB = 2000


def _tc1_body(x_ref, d0_ref, d1_ref, w_ref, z_ref, dinv_ref):
  deg = d0_ref[:, 0:1] + d1_ref[:, 0:1] + 1.0
  dinv = lax.rsqrt(deg)
  xw = jnp.dot(x_ref[:, :], w_ref[:, :], preferred_element_type=jnp.float32)
  z_ref[:, :] = xw * dinv
  dinv_ref[:, :] = jnp.broadcast_to(dinv, (_RB, 16))


def _tc_z1(x, d0, d1, W1):
  return pl.pallas_call(
      _tc1_body,
      grid=(N // _RB,),
      in_specs=[
          pl.BlockSpec((_RB, D), lambda i: (i, 0)),
          pl.BlockSpec((_RB, 1), lambda i: (i, 0)),
          pl.BlockSpec((_RB, 1), lambda i: (i, 0)),
          pl.BlockSpec((D, D), lambda i: (0, 0)),
      ],
      out_specs=[
          pl.BlockSpec((_RB, D), lambda i: (i, 0)),
          pl.BlockSpec((_RB, 16), lambda i: (i, 0)),
      ],
      out_shape=[
          jax.ShapeDtypeStruct((N, D), jnp.float32),
          jax.ShapeDtypeStruct((N, 16), jnp.float32),
      ],
  )(x, d0, d1, W1)


def _tc2_body(a0_ref, a1_ref, z1_ref, dinv_ref, b_ref, w_ref, z2_ref):
  dinv = dinv_ref[:, 0:1]
  h = (a0_ref[:, :] + a1_ref[:, :] + z1_ref[:, :]) * dinv + b_ref[:, :]
  h = jnp.maximum(h, 0.0)
  z2_ref[:, :] = jnp.dot(h, w_ref[:, :],
                         preferred_element_type=jnp.float32) * dinv


def _tc_mid(a0, a1, z1, dinv16, b1, W2):
  return pl.pallas_call(
      _tc2_body,
      grid=(N // _RB,),
      in_specs=[
          pl.BlockSpec((_RB, D), lambda i: (i, 0)),
          pl.BlockSpec((_RB, D), lambda i: (i, 0)),
          pl.BlockSpec((_RB, D), lambda i: (i, 0)),
          pl.BlockSpec((_RB, 16), lambda i: (i, 0)),
          pl.BlockSpec((1, D), lambda i: (0, 0)),
          pl.BlockSpec((D, D), lambda i: (0, 0)),
      ],
      out_specs=pl.BlockSpec((_RB, D), lambda i: (i, 0)),
      out_shape=jax.ShapeDtypeStruct((N, D), jnp.float32),
  )(a0, a1, z1, dinv16, b1, W2)


def _tc3_body(a0_ref, a1_ref, z2_ref, dinv_ref, b_ref, g_ref, beta_ref,
              out_ref):
  dinv = dinv_ref[:, 0:1]
  h = (a0_ref[:, :] + a1_ref[:, :] + z2_ref[:, :]) * dinv + b_ref[:, :]
  mu = jnp.mean(h, axis=-1, keepdims=True)
  hc = h - mu
  var = jnp.mean(hc * hc, axis=-1, keepdims=True)
  out_ref[:, :] = hc * lax.rsqrt(var + 1e-5) * g_ref[:, :] + beta_ref[:, :]


def _tc_final(a0, a1, z2, dinv16, b2, gamma, beta):
  return pl.pallas_call(
      _tc3_body,
      grid=(N // _RB,),
      in_specs=[
          pl.BlockSpec((_RB, D), lambda i: (i, 0)),
          pl.BlockSpec((_RB, D), lambda i: (i, 0)),
          pl.BlockSpec((_RB, D), lambda i: (i, 0)),
          pl.BlockSpec((_RB, 16), lambda i: (i, 0)),
          pl.BlockSpec((1, D), lambda i: (0, 0)),
          pl.BlockSpec((1, D), lambda i: (0, 0)),
          pl.BlockSpec((1, D), lambda i: (0, 0)),
      ],
      out_specs=pl.BlockSpec((_RB, D), lambda i: (i, 0)),
      out_shape=jax.ShapeDtypeStruct((N, D), jnp.float32),
  )(a0, a1, z2, dinv16, b2, gamma, beta)


def kernel(x, edge_index, W1, b1, W2, b2, gamma, beta):
  src = edge_index[0]
  dst = edge_index[1]
  padi = jnp.full((EPAD - E,), N, jnp.int32)
  src2 = jnp.concatenate([src, padi]).reshape(ECHUNKS, 128)
  dst2 = jnp.concatenate([dst, padi]).reshape(ECHUNKS, 128)

  degf = _sc_degree(dst2)[:, :, :5, :].reshape(NC, NACC)
  d0 = degf[0, :N].reshape(N, 1)
  d1 = degf[1, :N].reshape(N, 1)

  z1, dinv16 = _tc_z1(x, d0, d1, W1)

  zpad = jnp.zeros((NACC - N, D), jnp.float32)
  ag1 = _sc_aggregate(jnp.concatenate([z1, zpad]), src2, dst2)
  z2 = _tc_mid(ag1[0, :N], ag1[1, :N], z1, dinv16,
               b1.reshape(1, D), W2)

  ag2 = _sc_aggregate(jnp.concatenate([z2, zpad]), src2, dst2)
  return _tc_final(ag2[0, :N], ag2[1, :N], z2, dinv16,
                   b2.reshape(1, D), gamma.reshape(1, D),
                   beta.reshape(1, D))


# speedup vs baseline: 9.8025x; 1.1108x over previous
import functools

import jax
import jax.numpy as jnp
from jax import lax
from jax.experimental import pallas as pl
from jax.experimental.pallas import tpu as pltpu
from jax.experimental.pallas import tpu_sc as plsc

N = 10000
D = 128
E = 320000

NC = 2
NS = 16
NACC = 10240
RT = NACC // NS
CT = 80
EPAD = NC * NS * CT * 128
ECHUNKS = EPAD // 128

_MESH = plsc.VectorSubcoreMesh(core_axis_name="c", subcore_axis_name="s")


def _zero_fill(ref, nrows, ncols):
  zv = jnp.zeros((16,), jnp.float32)

  def row(i, _):
    def col(j, _):
      ref[i, pl.ds(j * 16, 16)] = zv
      return 0
    return lax.fori_loop(0, ncols // 16, col, 0)

  lax.fori_loop(0, nrows, row, 0)


@functools.partial(
    pl.kernel,
    out_type=jax.ShapeDtypeStruct((NC, NS, 8, 128), jnp.float32),
    mesh=_MESH,
    scratch_types=[
        pltpu.VMEM_SHARED((NACC, 128), jnp.float32),
        pltpu.VMEM((CT, 128), jnp.int32),
        pltpu.VMEM((128, 128), jnp.float32),
        pltpu.VMEM((128, 128), jnp.float32),
        pltpu.VMEM((8, 128), jnp.float32),
    ],
)
def _sc_degree(dst2_hbm, out_hbm, htab, idx_d, ones, zbuf, compact):
  c = lax.axis_index("c")
  s = lax.axis_index("s")
  wid = s * NC + c

  _zero_fill(ones, 128, 128)
  for t in range(RT // 128):
    pltpu.sync_copy(ones, htab.at[pl.ds(s * RT + t * 128, 128)])

  ov = jnp.ones((16,), jnp.float32)

  def orow(i, _):
    def ocol(j, _):
      ones[i, pl.ds(j * 16, 16)] = ov
      return 0
    return lax.fori_loop(0, 128 // 16, ocol, 0)
  lax.fori_loop(0, 128, orow, 0)
  plsc.subcore_barrier()

  pltpu.sync_copy(dst2_hbm.at[pl.ds(wid * CT, CT)], idx_d)

  def chunk(k, _):
    pltpu.sync_copy(ones, htab.at[idx_d.at[k]], add=True)
    return 0
  lax.fori_loop(0, CT, chunk, 0)

  plsc.subcore_barrier()
  iota16 = lax.iota(jnp.int32, 16)
  zf = jnp.zeros((16,), jnp.float32)
  for r in range(5, 8):
    for cc in range(8):
      compact[r, pl.ds(cc * 16, 16)] = zf
  for t in range(RT // 128):
    pltpu.sync_copy(htab.at[pl.ds(s * RT + t * 128, 128)], zbuf)
    for j8 in range(8):
      v = zf
      for i in range(16):
        rv = zbuf[j8 * 16 + i, pl.ds(0, 16)]
        v = v + jnp.where(iota16 == i, rv, 0.0)
      compact[t, pl.ds(j8 * 16, 16)] = v
  pltpu.sync_copy(compact, out_hbm.at[c, s])


@functools.partial(
    pl.kernel,
    out_type=jax.ShapeDtypeStruct((NC, NACC, D), jnp.float32),
    mesh=_MESH,
    scratch_types=[
        pltpu.VMEM_SHARED((NACC, D), jnp.float32),
        pltpu.VMEM((CT // 2, 128), jnp.int32),
        pltpu.VMEM((CT // 2, 128), jnp.int32),
        pltpu.VMEM((128, D), jnp.float32),
        pltpu.VMEM((128, D), jnp.float32),
        pltpu.SemaphoreType.DMA,
        pltpu.SemaphoreType.DMA,
    ],
)
def _sc_aggregate(z_hbm, src2_hbm, dst2_hbm, out_hbm,
                  acc, idx_s, idx_d, rows_a, rows_b, sem_a, sem_b):
  c = lax.axis_index("c")
  s = lax.axis_index("s")
  wid = s * NC + c

  pltpu.sync_copy(z_hbm.at[pl.ds(N, 128)], rows_a)
  for t in range(RT // 128):
    pltpu.sync_copy(rows_a, acc.at[pl.ds(s * RT + t * 128, 128)])
  plsc.subcore_barrier()

  H = CT // 2
  for half in range(2):
    base = wid * CT + half * H
    pltpu.sync_copy(src2_hbm.at[pl.ds(base, H)], idx_s)
    pltpu.sync_copy(dst2_hbm.at[pl.ds(base, H)], idx_d)
    pltpu.async_copy(z_hbm.at[idx_s.at[0]], rows_a, sem_a)

    def body(k, _):
      i0 = 2 * k
      pltpu.async_copy(z_hbm.at[idx_s.at[i0 + 1]], rows_b, sem_b)
      pltpu.make_async_copy(z_hbm.at[idx_s.at[i0]], rows_a, sem_a).wait()
      pltpu.sync_copy(rows_a, acc.at[idx_d.at[i0]], add=True)

      @pl.when(k < H // 2 - 1)
      def _():
        pltpu.async_copy(z_hbm.at[idx_s.at[i0 + 2]], rows_a, sem_a)

      pltpu.make_async_copy(z_hbm.at[idx_s.at[i0 + 1]], rows_b,
                            sem_b).wait()
      pltpu.sync_copy(rows_b, acc.at[idx_d.at[i0 + 1]], add=True)
      return 0

    lax.fori_loop(0, H // 2, body, 0)

  plsc.subcore_barrier()
  for t in range(RT // 128):
    pltpu.sync_copy(acc.at[pl.ds(s * RT + t * 128, 128)], rows_a)
    pltpu.sync_copy(rows_a, out_hbm.at[c, pl.ds(s * RT + t * 128, 128)])


_RB = 2000


def _tc1_body(x_ref, d0_ref, d1_ref, w_ref, z_ref, dinv_ref):
  deg = d0_ref[:, 0:1] + d1_ref[:, 0:1] + 1.0
  dinv = lax.rsqrt(deg)
  xw = jnp.dot(x_ref[:, :], w_ref[:, :], preferred_element_type=jnp.float32)
  z_ref[:, :] = xw * dinv
  dinv_ref[:, :] = jnp.broadcast_to(dinv, (_RB, 16))


def _tc_z1(x, d0, d1, W1):
  return pl.pallas_call(
      _tc1_body,
      grid=(N // _RB,),
      in_specs=[
          pl.BlockSpec((_RB, D), lambda i: (i, 0)),
          pl.BlockSpec((_RB, 1), lambda i: (i, 0)),
          pl.BlockSpec((_RB, 1), lambda i: (i, 0)),
          pl.BlockSpec((D, D), lambda i: (0, 0)),
      ],
      out_specs=[
          pl.BlockSpec((_RB, D), lambda i: (i, 0)),
          pl.BlockSpec((_RB, 16), lambda i: (i, 0)),
      ],
      out_shape=[
          jax.ShapeDtypeStruct((N, D), jnp.float32),
          jax.ShapeDtypeStruct((N, 16), jnp.float32),
      ],
  )(x, d0, d1, W1)


def _tc2_body(a0_ref, a1_ref, z1_ref, dinv_ref, b_ref, w_ref, z2_ref):
  dinv = dinv_ref[:, 0:1]
  h = (a0_ref[:, :] + a1_ref[:, :] + z1_ref[:, :]) * dinv + b_ref[:, :]
  h = jnp.maximum(h, 0.0)
  z2_ref[:, :] = jnp.dot(h, w_ref[:, :],
                         preferred_element_type=jnp.float32) * dinv


def _tc_mid(a0, a1, z1, dinv16, b1, W2):
  return pl.pallas_call(
      _tc2_body,
      grid=(N // _RB,),
      in_specs=[
          pl.BlockSpec((_RB, D), lambda i: (i, 0)),
          pl.BlockSpec((_RB, D), lambda i: (i, 0)),
          pl.BlockSpec((_RB, D), lambda i: (i, 0)),
          pl.BlockSpec((_RB, 16), lambda i: (i, 0)),
          pl.BlockSpec((1, D), lambda i: (0, 0)),
          pl.BlockSpec((D, D), lambda i: (0, 0)),
      ],
      out_specs=pl.BlockSpec((_RB, D), lambda i: (i, 0)),
      out_shape=jax.ShapeDtypeStruct((N, D), jnp.float32),
  )(a0, a1, z1, dinv16, b1, W2)


def _tc3_body(a0_ref, a1_ref, z2_ref, dinv_ref, b_ref, g_ref, beta_ref,
              out_ref):
  dinv = dinv_ref[:, 0:1]
  h = (a0_ref[:, :] + a1_ref[:, :] + z2_ref[:, :]) * dinv + b_ref[:, :]
  mu = jnp.mean(h, axis=-1, keepdims=True)
  hc = h - mu
  var = jnp.mean(hc * hc, axis=-1, keepdims=True)
  out_ref[:, :] = hc * lax.rsqrt(var + 1e-5) * g_ref[:, :] + beta_ref[:, :]


def _tc_final(a0, a1, z2, dinv16, b2, gamma, beta):
  return pl.pallas_call(
      _tc3_body,
      grid=(N // _RB,),
      in_specs=[
          pl.BlockSpec((_RB, D), lambda i: (i, 0)),
          pl.BlockSpec((_RB, D), lambda i: (i, 0)),
          pl.BlockSpec((_RB, D), lambda i: (i, 0)),
          pl.BlockSpec((_RB, 16), lambda i: (i, 0)),
          pl.BlockSpec((1, D), lambda i: (0, 0)),
          pl.BlockSpec((1, D), lambda i: (0, 0)),
          pl.BlockSpec((1, D), lambda i: (0, 0)),
      ],
      out_specs=pl.BlockSpec((_RB, D), lambda i: (i, 0)),
      out_shape=jax.ShapeDtypeStruct((N, D), jnp.float32),
  )(a0, a1, z2, dinv16, b2, gamma, beta)


def kernel(x, edge_index, W1, b1, W2, b2, gamma, beta):
  src = edge_index[0]
  dst = edge_index[1]
  padi = jnp.full((EPAD - E,), N, jnp.int32)
  src2 = jnp.concatenate([src, padi]).reshape(ECHUNKS, 128)
  dst2 = jnp.concatenate([dst, padi]).reshape(ECHUNKS, 128)

  degf = _sc_degree(dst2)[:, :, :5, :].reshape(NC, NACC)
  d0 = degf[0, :N].reshape(N, 1)
  d1 = degf[1, :N].reshape(N, 1)

  z1, dinv16 = _tc_z1(x, d0, d1, W1)

  zpad = jnp.zeros((NACC - N, D), jnp.float32)
  ag1 = _sc_aggregate(jnp.concatenate([z1, zpad]), src2, dst2)
  z2 = _tc_mid(ag1[0, :N], ag1[1, :N], z1, dinv16,
               b1.reshape(1, D), W2)

  ag2 = _sc_aggregate(jnp.concatenate([z2, zpad]), src2, dst2)
  return _tc_final(ag2[0, :N], ag2[1, :N], z2, dinv16,
                   b2.reshape(1, D), gamma.reshape(1, D),
                   beta.reshape(1, D))
